# baseline (speedup 1.0000x reference)
import functools

import jax
import jax.numpy as jnp
from jax import lax
from jax.experimental import pallas as pl
from jax.experimental.pallas import tpu as pltpu
from jax.experimental.pallas import tpu_sc as plsc

E = 320000
N = 10000
DF = 128
DE = 16
DU = 64

NC = 2
NS = 16
NW = NC * NS
ER = E // 8

IDXW = 128
IDX_ROWS = E // IDXW
MACRO_I = 4
MACRO_E = MACRO_I * IDXW
MACRO_R = MACRO_E // 8
SUB_R = IDXW // 8
NMAC = E // MACRO_E
FULL_W = NMAC - 19 * NW
NPT = N // NS

A_BLK = 4000


def _edge_dense_body(ea_ref, w_ref, c_ref, o_ref):
    o_ref[...] = (
        jnp.dot(ea_ref[...], w_ref[...], preferred_element_type=jnp.float32)
        + c_ref[...]
    )


def _edge_dense(ea2, w_bd, cvec8):
    return pl.pallas_call(
        _edge_dense_body,
        grid=(ER // A_BLK,),
        in_specs=[
            pl.BlockSpec((A_BLK, DF), lambda i: (i, 0)),
            pl.BlockSpec((DF, DF), lambda i: (0, 0)),
            pl.BlockSpec((1, DF), lambda i: (0, 0)),
        ],
        out_specs=pl.BlockSpec((A_BLK, DF), lambda i: (i, 0)),
        out_shape=jax.ShapeDtypeStruct((ER, DF), jnp.float32),
    )(ea2, w_bd, cvec8)


def _node_pre_body(x_ref, w23_ref, ue_ref, we4_ref, be_ref, x2_ref, x3_ref, cv_ref):
    x23 = jnp.dot(x_ref[...], w23_ref[...], preferred_element_type=jnp.float32)
    x2_ref[...] = x23[:, :DE]
    x3_ref[...] = x23[:, DE:]
    cv_ref[...] = (
        jnp.dot(ue_ref[...], we4_ref[...], preferred_element_type=jnp.float32)
        + be_ref[...]
    )


def _node_pre(x, w_e23, u_e, w_e4, b_edge):
    return pl.pallas_call(
        _node_pre_body,
        out_shape=(
            jax.ShapeDtypeStruct((N, DE), jnp.float32),
            jax.ShapeDtypeStruct((N, DE), jnp.float32),
            jax.ShapeDtypeStruct((1, DE), jnp.float32),
        ),
    )(x, w_e23, u_e, w_e4, b_edge)


_SC_MESH = plsc.VectorSubcoreMesh(core_axis_name="c", subcore_axis_name="s")


@functools.partial(
    pl.kernel,
    out_type=(
        jax.ShapeDtypeStruct((E, DE), jnp.float32),
        jax.ShapeDtypeStruct((NC, NS, NPT, DE), jnp.float32),
        jax.ShapeDtypeStruct((NC, NS, NPT, DE), jnp.float32),
    ),
    mesh=_SC_MESH,
    compiler_params=pltpu.CompilerParams(use_tc_tiling_on_sc=False),
    scratch_types=[
        pltpu.VMEM((MACRO_I, IDXW), jnp.int32),
        pltpu.VMEM((MACRO_I, IDXW), jnp.int32),
        pltpu.VMEM((MACRO_R, DF), jnp.float32),
        pltpu.VMEM((MACRO_E, DE), jnp.float32),
        pltpu.VMEM((MACRO_E, DE), jnp.float32),
        pltpu.VMEM((MACRO_E, DE), jnp.float32),
        pltpu.VMEM((NPT, DE), jnp.float32),
        pltpu.VMEM_SHARED((N, DE), jnp.float32),
        pltpu.VMEM_SHARED((N, DE), jnp.float32),
        pltpu.SemaphoreType.DMA,
        pltpu.SemaphoreType.DMA,
        pltpu.SemaphoreType.DMA,
    ],
)
def _sc_edges(a2_hbm, row_hbm, col_hbm, x2_hbm, x3_hbm,
              eout_hbm, sentp_hbm, recvp_hbm,
              rowv, colv, abuf, g2, g3, obuf, zbuf, sent_acc, recv_acc,
              semG, semA, semE):
    c = lax.axis_index("c")
    s = lax.axis_index("s")
    wid = c * NS + s

    zero = jnp.zeros((DE,), jnp.float32)

    def _zero_row(i, carry):
        zbuf[i] = zero
        return carry

    lax.fori_loop(0, NPT, _zero_row, 0)
    pltpu.sync_copy(zbuf, sent_acc.at[pl.ds(s * NPT, NPT)])
    pltpu.sync_copy(zbuf, recv_acc.at[pl.ds(s * NPT, NPT)])
    plsc.subcore_barrier()

    n_mac = jnp.where(wid < FULL_W, 20, 19)

    def _fire_sub(j):
        d = pl.ds(j * IDXW, IDXW)
        return (pltpu.async_copy(x2_hbm.at[rowv.at[j]], g2.at[d], semG),
                pltpu.async_copy(x3_hbm.at[colv.at[j]], g3.at[d], semG))

    def _macro(t, carry):
        k = wid + NW * t
        pltpu.sync_copy(row_hbm.at[pl.ds(k * MACRO_I, MACRO_I)], rowv)
        pltpu.sync_copy(col_hbm.at[pl.ds(k * MACRO_I, MACRO_I)], colv)
        acp = pltpu.async_copy(a2_hbm.at[pl.ds(k * MACRO_R, MACRO_R)], abuf, semA)
        cps = _fire_sub(0)
        acp.wait()
        for i in range(MACRO_I):
            for cp in cps:
                cp.wait()
            if i + 1 < MACRO_I:
                cps = _fire_sub(i + 1)

            def _row(pr, carry2):
                for j in range(8):
                    e = pr * 8 + j
                    obuf[e] = abuf[pr, pl.ds(DE * j, DE)] + g2[e] + g3[e]
                return carry2

            lax.fori_loop(i * SUB_R, (i + 1) * SUB_R, _row, 0)
            d = pl.ds(i * IDXW, IDXW)
            if i + 1 == MACRO_I:
                ecp = pltpu.async_copy(
                    obuf, eout_hbm.at[pl.ds(k * MACRO_E, MACRO_E)], semE)
            pltpu.sync_copy(obuf.at[d], sent_acc.at[rowv.at[i]], add=True)
            pltpu.sync_copy(obuf.at[d], recv_acc.at[colv.at[i]], add=True)
        ecp.wait()
        return carry

    lax.fori_loop(0, n_mac, _macro, 0)
    plsc.subcore_barrier()
    pltpu.sync_copy(sent_acc.at[pl.ds(s * NPT, NPT)], sentp_hbm.at[c, s])
    pltpu.sync_copy(recv_acc.at[pl.ds(s * NPT, NPT)], recvp_hbm.at[c, s])


def _node_glob_body(x_ref, sp_ref, rp_ref, wn1_ref, wn23_ref, un_ref, wn4_ref,
                    bn_ref, u_ref, wgu_ref, wgx_ref, wge_ref, bg_ref,
                    xn_ref, un_out_ref):
    sent = sp_ref[0] + sp_ref[1]
    recv = rp_ref[0] + rp_ref[1]
    sr = jnp.concatenate([sent, recv], axis=1)
    xn = (
        jnp.dot(x_ref[...], wn1_ref[...], preferred_element_type=jnp.float32)
        + jnp.dot(sr, wn23_ref[...], preferred_element_type=jnp.float32)
        + jnp.dot(un_ref[...], wn4_ref[...], preferred_element_type=jnp.float32)
        + bn_ref[...]
    )
    xn_ref[...] = xn
    node_sum = jnp.sum(xn, axis=0, keepdims=True)
    edge_sum = jnp.sum(sent, axis=0, keepdims=True)
    un_out_ref[...] = (
        jnp.dot(u_ref[...], wgu_ref[...], preferred_element_type=jnp.float32)
        + jnp.dot(node_sum, wgx_ref[...], preferred_element_type=jnp.float32)
        + jnp.dot(edge_sum, wge_ref[...], preferred_element_type=jnp.float32)
        + bg_ref[...]
    )


def _node_glob(x, sentp, recvp, wn1, wn23, u_n, wn4, b_node, u, wgu, wgx, wge, b_glob):
    return pl.pallas_call(
        _node_glob_body,
        out_shape=(
            jax.ShapeDtypeStruct((N, DF), jnp.float32),
            jax.ShapeDtypeStruct((1, DU), jnp.float32),
        ),
    )(x, sentp, recvp, wn1, wn23, u_n, wn4, b_node, u, wgu, wgx, wge, b_glob)


def kernel(x, edge_index, edge_attr, u, node_batch, edge_batch, num_nodes,
           num_edges, W_edge, b_edge, W_node, b_node, W_glob, b_glob):
    e_scale = jnp.asarray(num_edges - edge_attr.shape[0] + 1, dtype=u.dtype)
    n_scale = jnp.asarray(num_nodes - x.shape[0] + 1, dtype=u.dtype)
    u_e = u * e_scale
    u_n = u * n_scale

    row2 = edge_index[0].reshape(IDX_ROWS, IDXW)
    col2 = edge_index[1].reshape(IDX_ROWS, IDXW)
    ea2 = edge_attr.reshape(ER, DF)

    w_e1 = W_edge[:DE]
    w_bd = jnp.kron(jnp.eye(8, dtype=W_edge.dtype), w_e1)
    w_e23 = jnp.concatenate([W_edge[DE:DE + DF], W_edge[DE + DF:DE + 2 * DF]],
                            axis=1)
    w_e4 = W_edge[DE + 2 * DF:]

    x2, x3, cvec = _node_pre(x, w_e23, u_e, w_e4, b_edge.reshape(1, DE))
    cvec8 = jnp.tile(cvec, (1, 8))
    a2 = _edge_dense(ea2, w_bd, cvec8)
    eout, sentp, recvp = _sc_edges(a2, row2, col2, x2, x3)
    sentp = sentp.reshape(NC, N, DE)
    recvp = recvp.reshape(NC, N, DE)

    wn1 = W_node[:DF]
    wn23 = W_node[DF:DF + 2 * DE]
    wn4 = W_node[DF + 2 * DE:]
    wgu = W_glob[:DU]
    wgx = W_glob[DU:DU + DF]
    wge = W_glob[DU + DF:]

    xn, un = _node_glob(x, sentp, recvp, wn1, wn23, u_n, wn4,
                        b_node.reshape(1, DF), u, wgu, wgx, wge,
                        b_glob.reshape(1, DU))
    return (xn, eout, un)


# speedup vs baseline: 1.0277x; 1.0277x over previous
import functools

import jax
import jax.numpy as jnp
from jax import lax
from jax.experimental import pallas as pl
from jax.experimental.pallas import tpu as pltpu
from jax.experimental.pallas import tpu_sc as plsc

E = 320000
N = 10000
DF = 128
DE = 16
DU = 64

NC = 2
NS = 16
NW = NC * NS
ER = E // 8

IDXW = 128
IDX_ROWS = E // IDXW
MACRO_I = 4
MACRO_E = MACRO_I * IDXW
MACRO_R = MACRO_E // 8
SUB_R = IDXW // 8
NMAC = E // MACRO_E
FULL_W = NMAC - 19 * NW
NPT = N // NS

A_BLK = 4000


def _edge_dense_body(ea_ref, w_ref, c_ref, o_ref):
    o_ref[...] = (
        jnp.dot(ea_ref[...], w_ref[...], preferred_element_type=jnp.float32)
        + c_ref[...]
    )


def _edge_dense(ea2, w_bd, cvec8):
    return pl.pallas_call(
        _edge_dense_body,
        grid=(ER // A_BLK,),
        in_specs=[
            pl.BlockSpec((A_BLK, DF), lambda i: (i, 0)),
            pl.BlockSpec((DF, DF), lambda i: (0, 0)),
            pl.BlockSpec((1, DF), lambda i: (0, 0)),
        ],
        out_specs=pl.BlockSpec((A_BLK, DF), lambda i: (i, 0)),
        out_shape=jax.ShapeDtypeStruct((ER, DF), jnp.float32),
    )(ea2, w_bd, cvec8)


N8 = N // 8


def _node_pre_body(xp_ref, k2_ref, k3_ref, ue_ref, we4_ref, be_ref,
                   x2_ref, x3_ref, cv_ref):
    xp = xp_ref[...]
    x2_ref[...] = jnp.dot(xp, k2_ref[...], preferred_element_type=jnp.float32)
    x3_ref[...] = jnp.dot(xp, k3_ref[...], preferred_element_type=jnp.float32)
    cv_ref[...] = (
        jnp.dot(ue_ref[...], we4_ref[...], preferred_element_type=jnp.float32)
        + be_ref[...]
    )


def _node_pre(xp, k2, k3, u_e, w_e4, b_edge):
    return pl.pallas_call(
        _node_pre_body,
        out_shape=(
            jax.ShapeDtypeStruct((N8, DF), jnp.float32),
            jax.ShapeDtypeStruct((N8, DF), jnp.float32),
            jax.ShapeDtypeStruct((1, DE), jnp.float32),
        ),
    )(xp, k2, k3, u_e, w_e4, b_edge)


_SC_MESH = plsc.VectorSubcoreMesh(core_axis_name="c", subcore_axis_name="s")


@functools.partial(
    pl.kernel,
    out_type=(
        jax.ShapeDtypeStruct((E, DE), jnp.float32),
        jax.ShapeDtypeStruct((NC, NS, NPT, DE), jnp.float32),
        jax.ShapeDtypeStruct((NC, NS, NPT, DE), jnp.float32),
    ),
    mesh=_SC_MESH,
    compiler_params=pltpu.CompilerParams(use_tc_tiling_on_sc=False),
    scratch_types=[
        pltpu.VMEM((MACRO_I, IDXW), jnp.int32),
        pltpu.VMEM((MACRO_I, IDXW), jnp.int32),
        pltpu.VMEM((MACRO_R, DF), jnp.float32),
        pltpu.VMEM((MACRO_E, DE), jnp.float32),
        pltpu.VMEM((MACRO_E, DE), jnp.float32),
        pltpu.VMEM((MACRO_E, DE), jnp.float32),
        pltpu.VMEM((NPT, DE), jnp.float32),
        pltpu.VMEM_SHARED((N, DE), jnp.float32),
        pltpu.VMEM_SHARED((N, DE), jnp.float32),
        pltpu.SemaphoreType.DMA,
        pltpu.SemaphoreType.DMA,
        pltpu.SemaphoreType.DMA,
    ],
)
def _sc_edges(a2_hbm, row_hbm, col_hbm, x2_hbm, x3_hbm,
              eout_hbm, sentp_hbm, recvp_hbm,
              rowv, colv, abuf, g2, g3, obuf, zbuf, sent_acc, recv_acc,
              semG, semA, semE):
    c = lax.axis_index("c")
    s = lax.axis_index("s")
    wid = c * NS + s

    zero = jnp.zeros((DE,), jnp.float32)

    def _zero_row(i, carry):
        zbuf[i] = zero
        return carry

    lax.fori_loop(0, NPT, _zero_row, 0)
    pltpu.sync_copy(zbuf, sent_acc.at[pl.ds(s * NPT, NPT)])
    pltpu.sync_copy(zbuf, recv_acc.at[pl.ds(s * NPT, NPT)])
    plsc.subcore_barrier()

    n_mac = jnp.where(wid < FULL_W, 20, 19)

    def _fire_sub(j):
        d = pl.ds(j * IDXW, IDXW)
        return (pltpu.async_copy(x2_hbm.at[rowv.at[j]], g2.at[d], semG),
                pltpu.async_copy(x3_hbm.at[colv.at[j]], g3.at[d], semG))

    def _macro(t, carry):
        k = wid + NW * t
        pltpu.sync_copy(row_hbm.at[pl.ds(k * MACRO_I, MACRO_I)], rowv)
        pltpu.sync_copy(col_hbm.at[pl.ds(k * MACRO_I, MACRO_I)], colv)
        acp = pltpu.async_copy(a2_hbm.at[pl.ds(k * MACRO_R, MACRO_R)], abuf, semA)
        cps = _fire_sub(0)
        acp.wait()
        for i in range(MACRO_I):
            for cp in cps:
                cp.wait()
            if i + 1 < MACRO_I:
                cps = _fire_sub(i + 1)

            def _row(pr, carry2):
                for j in range(8):
                    e = pr * 8 + j
                    obuf[e] = abuf[pr, pl.ds(DE * j, DE)] + g2[e] + g3[e]
                return carry2

            lax.fori_loop(i * SUB_R, (i + 1) * SUB_R, _row, 0)
            d = pl.ds(i * IDXW, IDXW)
            if i + 1 == MACRO_I:
                ecp = pltpu.async_copy(
                    obuf, eout_hbm.at[pl.ds(k * MACRO_E, MACRO_E)], semE)
            pltpu.sync_copy(obuf.at[d], sent_acc.at[rowv.at[i]], add=True)
            pltpu.sync_copy(obuf.at[d], recv_acc.at[colv.at[i]], add=True)
        ecp.wait()
        return carry

    lax.fori_loop(0, n_mac, _macro, 0)
    plsc.subcore_barrier()
    pltpu.sync_copy(sent_acc.at[pl.ds(s * NPT, NPT)], sentp_hbm.at[c, s])
    pltpu.sync_copy(recv_acc.at[pl.ds(s * NPT, NPT)], recvp_hbm.at[c, s])


def _node_glob_body(xp_ref, sp_ref, rp_ref, wbd_ref, k2_ref, k3_ref, un_ref,
                    wn4_ref, bn_ref, u_ref, wgu_ref, wgx_ref, wge_ref, bg_ref,
                    xn_ref, un_out_ref):
    s2 = sp_ref[0] + sp_ref[1]
    r2 = rp_ref[0] + rp_ref[1]
    unt = (
        jnp.dot(un_ref[...], wn4_ref[...], preferred_element_type=jnp.float32)
        + bn_ref[...]
    )
    unt8 = jnp.concatenate([unt] * 8, axis=1)
    xn = (
        jnp.dot(xp_ref[...], wbd_ref[...], preferred_element_type=jnp.float32)
        + jnp.dot(s2, k2_ref[...], preferred_element_type=jnp.float32)
        + jnp.dot(r2, k3_ref[...], preferred_element_type=jnp.float32)
        + unt8
    )
    xn_ref[...] = xn
    ns8 = jnp.sum(xn, axis=0, keepdims=True)
    node_sum = sum(ns8[:, DF * j:DF * (j + 1)] for j in range(8))
    es8 = jnp.sum(s2, axis=0, keepdims=True)
    edge_sum = sum(es8[:, DE * j:DE * (j + 1)] for j in range(8))
    un_out_ref[...] = (
        jnp.dot(u_ref[...], wgu_ref[...], preferred_element_type=jnp.float32)
        + jnp.dot(node_sum, wgx_ref[...], preferred_element_type=jnp.float32)
        + jnp.dot(edge_sum, wge_ref[...], preferred_element_type=jnp.float32)
        + bg_ref[...]
    )


def _node_glob(xp, sentp2, recvp2, wbd_n, k2n, k3n, u_n, wn4, b_node, u,
               wgu, wgx, wge, b_glob):
    return pl.pallas_call(
        _node_glob_body,
        out_shape=(
            jax.ShapeDtypeStruct((N8, 8 * DF), jnp.float32),
            jax.ShapeDtypeStruct((1, DU), jnp.float32),
        ),
    )(xp, sentp2, recvp2, wbd_n, k2n, k3n, u_n, wn4, b_node, u,
      wgu, wgx, wge, b_glob)


def kernel(x, edge_index, edge_attr, u, node_batch, edge_batch, num_nodes,
           num_edges, W_edge, b_edge, W_node, b_node, W_glob, b_glob):
    e_scale = jnp.asarray(num_edges - edge_attr.shape[0] + 1, dtype=u.dtype)
    n_scale = jnp.asarray(num_nodes - x.shape[0] + 1, dtype=u.dtype)
    u_e = u * e_scale
    u_n = u * n_scale

    row2 = edge_index[0].reshape(IDX_ROWS, IDXW)
    col2 = edge_index[1].reshape(IDX_ROWS, IDXW)
    ea2 = edge_attr.reshape(ER, DF)
    xp = x.reshape(N8, 8 * DF)

    eye8 = jnp.eye(8, dtype=W_edge.dtype)
    w_e1 = W_edge[:DE]
    w_bd = jnp.kron(eye8, w_e1)
    k2e = jnp.kron(eye8, W_edge[DE:DE + DF])
    k3e = jnp.kron(eye8, W_edge[DE + DF:DE + 2 * DF])
    w_e4 = W_edge[DE + 2 * DF:]

    x2p, x3p, cvec = _node_pre(xp, k2e, k3e, u_e, w_e4, b_edge.reshape(1, DE))
    cvec8 = jnp.tile(cvec, (1, 8))
    a2 = _edge_dense(ea2, w_bd, cvec8)
    eout, sentp, recvp = _sc_edges(a2, row2, col2,
                                   x2p.reshape(N, DE), x3p.reshape(N, DE))
    sentp2 = sentp.reshape(NC, N8, DF)
    recvp2 = recvp.reshape(NC, N8, DF)

    wbd_n = jnp.kron(eye8, W_node[:DF])
    k2n = jnp.kron(eye8, W_node[DF:DF + DE])
    k3n = jnp.kron(eye8, W_node[DF + DE:DF + 2 * DE])
    wn4 = W_node[DF + 2 * DE:]
    wgu = W_glob[:DU]
    wgx = W_glob[DU:DU + DF]
    wge = W_glob[DU + DF:]

    xnp, un = _node_glob(xp, sentp2, recvp2, wbd_n, k2n, k3n, u_n, wn4,
                         b_node.reshape(1, DF), u, wgu, wgx, wge,
                         b_glob.reshape(1, DU))
    return (xnp.reshape(N, DF), eout, un)
